# baseline (speedup 1.0000x reference)
import functools

import jax
import jax.numpy as jnp
from jax import lax
from jax.experimental import pallas as pl
from jax.experimental.pallas import tpu as pltpu
from jax.experimental.pallas import tpu_sc as plsc

_MAX_ROWS = 30
_MAX_COLS = 30
_NUM_COLORS = 10
_HIDDEN = 512
_EPS = 1e-5
_NVALID = _MAX_ROWS * _MAX_COLS * _NUM_COLORS
_NINV = _MAX_ROWS * _MAX_COLS
_ROWS_PER_R = _MAX_COLS * _NUM_COLORS


def _ln(x, g, b):
    mu = jnp.mean(x, axis=-1, keepdims=True)
    xc = x - mu
    var = jnp.mean(xc * xc, axis=-1, keepdims=True)
    return xc * lax.rsqrt(var + _EPS) * g + b


def _build_tables_body(tbl_ref, row_ref, cc_ref, colb_ref, gamma_ref, beta_ref,
                       outv_ref, outi_ref):
    g = gamma_ref[...]
    b = beta_ref[...]
    outv_ref[...] = _ln(tbl_ref[...] + row_ref[...] + cc_ref[...], g, b)
    outi_ref[...] = _ln(row_ref[...] + colb_ref[...], g, b)


def _build_fused_tables(tbl, row_table, cc, colb, gamma2, beta2):
    tbl3 = tbl.reshape(_MAX_ROWS, _ROWS_PER_R, _HIDDEN)
    row3 = row_table.reshape(_MAX_ROWS, 1, _HIDDEN)
    outv, outi = pl.pallas_call(
        _build_tables_body,
        grid=(_MAX_ROWS,),
        in_specs=[
            pl.BlockSpec((1, _ROWS_PER_R, _HIDDEN), lambda i: (i, 0, 0)),
            pl.BlockSpec((1, 1, _HIDDEN), lambda i: (i, 0, 0)),
            pl.BlockSpec((_ROWS_PER_R, _HIDDEN), lambda i: (0, 0)),
            pl.BlockSpec((_MAX_COLS, _HIDDEN), lambda i: (0, 0)),
            pl.BlockSpec((1, _HIDDEN), lambda i: (0, 0)),
            pl.BlockSpec((1, _HIDDEN), lambda i: (0, 0)),
        ],
        out_specs=[
            pl.BlockSpec((1, _ROWS_PER_R, _HIDDEN), lambda i: (i, 0, 0)),
            pl.BlockSpec((1, _MAX_COLS, _HIDDEN), lambda i: (i, 0, 0)),
        ],
        out_shape=[
            jax.ShapeDtypeStruct((_MAX_ROWS, _ROWS_PER_R, _HIDDEN), jnp.float32),
            jax.ShapeDtypeStruct((_MAX_ROWS, _MAX_COLS, _HIDDEN), jnp.float32),
        ],
    )(tbl3, row3, cc, colb, gamma2, beta2)
    return outv.reshape(_NVALID, _HIDDEN), outi.reshape(_NINV, _HIDDEN)


def _sc_gather(fused, idxp, bsz, hw):
    info = plsc.get_sparse_core_info()
    nc, ns = info.num_cores, info.num_subcores
    nw = nc * ns
    d = fused.shape[1]
    ch = 96
    nb = 2
    cpb = (hw + ch - 1) // ch
    tail = hw - (cpb - 1) * ch
    bw = bsz // nw
    ipw = bw * cpb * ch
    steps = bw * cpb
    groups = steps // nb
    mesh = plsc.VectorSubcoreMesh(core_axis_name="c", subcore_axis_name="s")

    hwp = ((hw + 7) // 8) * 8
    tailw = ((tail + 7) // 8) * 8

    @functools.partial(
        pl.kernel,
        mesh=mesh,
        out_type=jax.ShapeDtypeStruct((bsz * hwp, d), jnp.float32),
        scratch_types=(
            [pltpu.VMEM((ipw,), jnp.int32)]
            + [pltpu.VMEM((ch, d), jnp.float32) for _ in range(nb)]
            + [pltpu.SemaphoreType.DMA for _ in range(2 * nb)]
        ),
    )
    def k(fused_hbm, idx_hbm, out_hbm, idx_v, *rest):
        bufs = rest[:nb]
        gsems = rest[nb:2 * nb]
        osems = rest[2 * nb:]
        wid = lax.axis_index("s") * nc + lax.axis_index("c")
        batch0 = wid * bw
        pltpu.sync_copy(idx_hbm.at[pl.ds(wid * ipw, ipw)], idx_v)

        def gather_desc(s, b):
            return pltpu.make_async_copy(
                fused_hbm.at[idx_v.at[pl.ds(s * ch, ch)]], bufs[b], gsems[b])

        def out_full(s, b):
            j = s // cpb
            c = s % cpb
            return pltpu.make_async_copy(
                bufs[b], out_hbm.at[pl.ds((batch0 + j) * hwp + c * ch, ch)],
                osems[b])

        def out_tail(s, b):
            j = s // cpb
            return pltpu.make_async_copy(
                bufs[b].at[pl.ds(0, tailw)],
                out_hbm.at[pl.ds((batch0 + j) * hwp + (cpb - 1) * ch, tailw)],
                osems[b])

        for b in range(nb):
            gather_desc(b, b).start()

        def body(t, carry):
            s0 = t * nb
            for b in range(nb):
                s = s0 + b
                is_tail = s % cpb == cpb - 1
                gather_desc(s, b).wait()

                @pl.when(jnp.logical_not(is_tail))
                def _():
                    out_full(s, b).start()

                @pl.when(is_tail)
                def _():
                    out_tail(s, b).start()
            for b in range(nb):
                s = s0 + b
                is_tail = s % cpb == cpb - 1

                @pl.when(jnp.logical_not(is_tail))
                def _():
                    out_full(s, b).wait()

                @pl.when(is_tail)
                def _():
                    out_tail(s, b).wait()

                @pl.when(s + nb < steps)
                def _():
                    gather_desc(s + nb, b).start()
            return carry

        lax.fori_loop(0, groups, body, 0)

    out2 = k(fused, idxp)
    return out2.reshape(bsz, hwp, d)[:, :hw, :]


def kernel(color_grid, valid_mask, coord_color_table, row_table, col_table,
           W_color, b_color, W_valid, b_valid, ln_gamma, ln_beta):
    bsz, h, w = color_grid.shape
    f32 = jnp.float32

    bias = (b_color + b_valid).astype(f32)
    wc_rows = W_color.T.astype(f32)
    wv_row = W_valid[:, 0].astype(f32)
    cc = (col_table[:, None, :] + wc_rows[None, :, :]).reshape(_ROWS_PER_R, _HIDDEN)
    cc = cc + (wv_row + bias)[None, :]
    colb = col_table + bias[None, :]
    gamma2 = ln_gamma.reshape(1, _HIDDEN).astype(f32)
    beta2 = ln_beta.reshape(1, _HIDDEN).astype(f32)
    tbl = coord_color_table[:_NVALID].astype(f32)

    fused_v, fused_i = _build_fused_tables(tbl, row_table.astype(f32), cc, colb,
                                           gamma2, beta2)
    fused = jnp.concatenate([fused_v, fused_i], axis=0)

    r_ids = jnp.arange(h, dtype=jnp.int32)
    c_ids = jnp.arange(w, dtype=jnp.int32)
    token = (r_ids[None, :, None] * (_MAX_COLS * _NUM_COLORS)
             + c_ids[None, None, :] * _NUM_COLORS
             + color_grid.astype(jnp.int32))
    inv = _NVALID + r_ids[None, :, None] * _MAX_COLS + c_ids[None, None, :]
    idx = jnp.where(valid_mask, token, inv).astype(jnp.int32).reshape(bsz, h * w)

    ch = 96
    cpb = (h * w + ch - 1) // ch
    pad = cpb * ch - h * w
    idxp = jnp.concatenate(
        [idx, jnp.zeros((bsz, pad), jnp.int32)], axis=1).reshape(-1)

    return _sc_gather(fused, idxp, bsz, h * w)


# speedup vs baseline: 2.2973x; 2.2973x over previous
import functools

import jax
import jax.numpy as jnp
from jax import lax
from jax.experimental import pallas as pl
from jax.experimental.pallas import tpu as pltpu
from jax.experimental.pallas import tpu_sc as plsc

_MAX_ROWS = 30
_MAX_COLS = 30
_NUM_COLORS = 10
_HIDDEN = 512
_EPS = 1e-5
_NVALID = _MAX_ROWS * _MAX_COLS * _NUM_COLORS
_NINV = _MAX_ROWS * _MAX_COLS
_ROWS_PER_R = _MAX_COLS * _NUM_COLORS


def _ln(x, g, b):
    mu = jnp.mean(x, axis=-1, keepdims=True)
    xc = x - mu
    var = jnp.mean(xc * xc, axis=-1, keepdims=True)
    return xc * lax.rsqrt(var + _EPS) * g + b


def _build_tables_body(tbl_ref, row_ref, cc_ref, colb_ref, gamma_ref, beta_ref,
                       outv_ref, outi_ref):
    g = gamma_ref[...]
    b = beta_ref[...]
    outv_ref[...] = _ln(tbl_ref[...] + row_ref[...] + cc_ref[...], g, b)
    outi_ref[...] = _ln(row_ref[...] + colb_ref[...], g, b)


def _build_fused_tables(tbl, row_table, cc, colb, gamma2, beta2):
    tbl3 = tbl.reshape(_MAX_ROWS, _ROWS_PER_R, _HIDDEN)
    row3 = row_table.reshape(_MAX_ROWS, 1, _HIDDEN)
    outv, outi = pl.pallas_call(
        _build_tables_body,
        grid=(_MAX_ROWS,),
        in_specs=[
            pl.BlockSpec((1, _ROWS_PER_R, _HIDDEN), lambda i: (i, 0, 0)),
            pl.BlockSpec((1, 1, _HIDDEN), lambda i: (i, 0, 0)),
            pl.BlockSpec((_ROWS_PER_R, _HIDDEN), lambda i: (0, 0)),
            pl.BlockSpec((_MAX_COLS, _HIDDEN), lambda i: (0, 0)),
            pl.BlockSpec((1, _HIDDEN), lambda i: (0, 0)),
            pl.BlockSpec((1, _HIDDEN), lambda i: (0, 0)),
        ],
        out_specs=[
            pl.BlockSpec((1, _ROWS_PER_R, _HIDDEN), lambda i: (i, 0, 0)),
            pl.BlockSpec((1, _MAX_COLS, _HIDDEN), lambda i: (i, 0, 0)),
        ],
        out_shape=[
            jax.ShapeDtypeStruct((_MAX_ROWS, _ROWS_PER_R, _HIDDEN), jnp.float32),
            jax.ShapeDtypeStruct((_MAX_ROWS, _MAX_COLS, _HIDDEN), jnp.float32),
        ],
    )(tbl3, row3, cc, colb, gamma2, beta2)
    return outv.reshape(_NVALID, _HIDDEN), outi.reshape(_NINV, _HIDDEN)


def _sc_gather(fused, idxp, bsz, hw, hwp):
    info = plsc.get_sparse_core_info()
    nc, ns = info.num_cores, info.num_subcores
    nw = nc * ns
    d = fused.shape[1]
    n = bsz * hwp
    bpw = n // nw
    ch = 64
    nb = 2
    steps = bpw // ch
    groups = steps // nb
    mesh = plsc.VectorSubcoreMesh(core_axis_name="c", subcore_axis_name="s")

    @functools.partial(
        pl.kernel,
        mesh=mesh,
        out_type=jax.ShapeDtypeStruct((n, d), jnp.float32),
        scratch_types=(
            [pltpu.VMEM((bpw,), jnp.int32)]
            + [pltpu.VMEM((ch, d), jnp.float32) for _ in range(nb)]
            + [pltpu.SemaphoreType.DMA for _ in range(2 * nb)]
        ),
    )
    def k(fused_hbm, idx_hbm, out_hbm, idx_v, *rest):
        bufs = rest[:nb]
        gsems = rest[nb:2 * nb]
        osems = rest[2 * nb:]
        wid = lax.axis_index("s") * nc + lax.axis_index("c")
        base = wid * bpw
        pltpu.sync_copy(idx_hbm.at[pl.ds(base, bpw)], idx_v)

        def gather_desc(g, b):
            return pltpu.make_async_copy(
                fused_hbm.at[idx_v.at[pl.ds(g * ch, ch)]], bufs[b], gsems[b])

        def out_desc(g, b):
            return pltpu.make_async_copy(
                bufs[b], out_hbm.at[pl.ds(base + g * ch, ch)], osems[b])

        for b in range(nb):
            gather_desc(b, b).start()

        def body(t, carry):
            g0 = t * nb
            for b in range(nb):
                gather_desc(g0 + b, b).wait()
                out_desc(g0 + b, b).start()
            for b in range(nb):
                out_desc(g0 + b, b).wait()

                @pl.when(g0 + nb + b < steps)
                def _():
                    gather_desc(g0 + nb + b, b).start()
            return carry

        lax.fori_loop(0, groups, body, 0)

    out2 = k(fused, idxp)
    return out2.reshape(bsz, hwp, d)[:, :hw, :]


def kernel(color_grid, valid_mask, coord_color_table, row_table, col_table,
           W_color, b_color, W_valid, b_valid, ln_gamma, ln_beta):
    bsz, h, w = color_grid.shape
    f32 = jnp.float32

    bias = (b_color + b_valid).astype(f32)
    wc_rows = W_color.T.astype(f32)
    wv_row = W_valid[:, 0].astype(f32)
    cc = (col_table[:, None, :] + wc_rows[None, :, :]).reshape(_ROWS_PER_R, _HIDDEN)
    cc = cc + (wv_row + bias)[None, :]
    colb = col_table + bias[None, :]
    gamma2 = ln_gamma.reshape(1, _HIDDEN).astype(f32)
    beta2 = ln_beta.reshape(1, _HIDDEN).astype(f32)
    tbl = coord_color_table[:_NVALID].astype(f32)

    fused_v, fused_i = _build_fused_tables(tbl, row_table.astype(f32), cc, colb,
                                           gamma2, beta2)
    fused = jnp.concatenate([fused_v, fused_i], axis=0)

    r_ids = jnp.arange(h, dtype=jnp.int32)
    c_ids = jnp.arange(w, dtype=jnp.int32)
    token = (r_ids[None, :, None] * (_MAX_COLS * _NUM_COLORS)
             + c_ids[None, None, :] * _NUM_COLORS
             + color_grid.astype(jnp.int32))
    inv = _NVALID + r_ids[None, :, None] * _MAX_COLS + c_ids[None, None, :]
    idx = jnp.where(valid_mask, token, inv).astype(jnp.int32).reshape(bsz, h * w)

    hw = h * w
    hwp = ((hw + 7) // 8) * 8
    idxp = jnp.concatenate(
        [idx, jnp.zeros((bsz, hwp - hw), jnp.int32)], axis=1).reshape(-1)

    return _sc_gather(fused, idxp, bsz, hw, hwp)
